# baseline (speedup 1.0000x reference)
import functools

import numpy as np
import jax
import jax.numpy as jnp
from jax import lax
from jax.experimental import pallas as pl
from jax.experimental.pallas import tpu as pltpu
import jax.experimental.pallas.tpu_sc as plsc

B, L, N, D, V, P = 1024, 50, 12, 64, 100000, 200

NW = 32
NB_ROWS = B * L * N
H_ROWS = B * L
NB_PER_W = NB_ROWS // NW
H_PER_W = H_ROWS // NW
NB_CHUNK = 120
NB_NCHUNK = NB_PER_W // NB_CHUNK
NB_RING = 8
NB_GROUPS = NB_NCHUNK // NB_RING
H_CHUNK = 80
H_NCHUNK = H_PER_W // H_CHUNK
H_RING = 4


def _pair_table_body(pb_ref, pa_ref, pio_ref, out_ref):
    comb = (pb_ref[...][:, None, :] + pa_ref[...][None, :, :]
            + pio_ref[1, :][None, None, :])
    out_ref[...] = comb.reshape(8 * P, D)


def _build_pair_table(pb, pa, pio):
    return pl.pallas_call(
        _pair_table_body,
        grid=(P // 8,),
        in_specs=[
            pl.BlockSpec((8, D), lambda i: (i, 0)),
            pl.BlockSpec((P, D), lambda i: (0, 0)),
            pl.BlockSpec((2, D), lambda i: (0, 0)),
        ],
        out_specs=pl.BlockSpec((8 * P, D), lambda i: (i, 0)),
        out_shape=jax.ShapeDtypeStruct((P * P, D), jnp.float32),
    )(pb, pa, pio)


def _sc_gather_body(emb_hbm, pair_hbm, idx_nb_hbm, idx_pr_hbm, idx_h_hbm,
                    dst_nb_hbm, feat_hbm, h_hbm,
                    idx_nb_v, idx_pr_v, idx_h_v, dst_nb_v, bufs,
                    gsem, asem, wsem):
    wid = lax.axis_index("s") * 2 + lax.axis_index("c")

    pltpu.sync_copy(idx_nb_hbm.at[wid], idx_nb_v)
    pltpu.sync_copy(idx_pr_hbm.at[wid], idx_pr_v)
    pltpu.sync_copy(idx_h_hbm.at[wid], idx_h_v)
    pltpu.sync_copy(dst_nb_hbm.at[wid], dst_nb_v)

    nb_base = wid * NB_PER_W
    h_base = wid * H_PER_W

    def h_gather(c, b):
        return pltpu.async_copy(
            emb_hbm.at[idx_h_v.at[c]], bufs.at[b, pl.ds(0, H_CHUNK)],
            gsem.at[b])

    def h_write(c, b):
        return pltpu.async_copy(
            bufs.at[b, pl.ds(0, H_CHUNK)],
            h_hbm.at[pl.ds(h_base + c * H_CHUNK, H_CHUNK)],
            wsem.at[b])

    for c in range(H_RING):
        h_gather(c, c)
    for c in range(H_NCHUNK):
        b = c % H_RING
        pltpu.make_async_copy(emb_hbm.at[idx_h_v.at[c]],
                              bufs.at[b, pl.ds(0, H_CHUNK)], gsem.at[b]).wait()
        h_write(c, b)
        if c + H_RING < H_NCHUNK:
            pltpu.make_async_copy(bufs.at[b, pl.ds(0, H_CHUNK)],
                                  h_hbm.at[pl.ds(h_base + c * H_CHUNK, H_CHUNK)],
                                  wsem.at[b]).wait()
            h_gather(c + H_RING, b)
    for c in range(H_NCHUNK - H_RING, H_NCHUNK):
        b = c % H_RING
        pltpu.make_async_copy(bufs.at[b, pl.ds(0, H_CHUNK)],
                              h_hbm.at[pl.ds(h_base + c * H_CHUNK, H_CHUNK)],
                              wsem.at[b]).wait()

    def nb_gather(c, j):
        return pltpu.async_copy(emb_hbm.at[idx_nb_v.at[c]], bufs.at[j],
                                gsem.at[j])

    for j in range(NB_RING):
        nb_gather(j, j)

    def group(g, _):
        for j in range(NB_RING):
            c = g * NB_RING + j
            pltpu.make_async_copy(emb_hbm.at[idx_nb_v.at[c]], bufs.at[j],
                                  gsem.at[j]).wait()
            pltpu.async_copy(pair_hbm.at[idx_pr_v.at[c]], bufs.at[j],
                             asem.at[j], add=True)
        for j in range(NB_RING):
            c = g * NB_RING + j
            pltpu.make_async_copy(pair_hbm.at[idx_pr_v.at[c]], bufs.at[j],
                                  asem.at[j]).wait()
            pltpu.async_copy(bufs.at[j], feat_hbm.at[dst_nb_v.at[c]],
                             wsem.at[j])
        for j in range(NB_RING):
            c = g * NB_RING + j
            pltpu.make_async_copy(bufs.at[j], feat_hbm.at[dst_nb_v.at[c]],
                                  wsem.at[j]).wait()

            @pl.when(g < NB_GROUPS - 1)
            def _():
                nb_gather(g * NB_RING + NB_RING + j, j)

        return 0

    lax.fori_loop(0, NB_GROUPS, group, 0)


def _sc_gather(emb, pair_tab, idx_nb, idx_pr, idx_h, dst_nb):
    mesh = plsc.VectorSubcoreMesh(core_axis_name="c", subcore_axis_name="s")
    fn = pl.kernel(
        _sc_gather_body,
        out_type=(
            jax.ShapeDtypeStruct((NB_ROWS, D), jnp.float32),
            jax.ShapeDtypeStruct((H_ROWS, D), jnp.float32),
        ),
        mesh=mesh,
        compiler_params=pltpu.CompilerParams(use_tc_tiling_on_sc=False),
        scratch_types=[
            pltpu.VMEM((NB_NCHUNK, NB_CHUNK), jnp.int32),
            pltpu.VMEM((NB_NCHUNK, NB_CHUNK), jnp.int32),
            pltpu.VMEM((H_NCHUNK, H_CHUNK), jnp.int32),
            pltpu.VMEM((NB_NCHUNK, NB_CHUNK), jnp.int32),
            pltpu.VMEM((NB_RING, NB_CHUNK, D), jnp.float32),
            pltpu.SemaphoreType.DMA((NB_RING,)),
            pltpu.SemaphoreType.DMA((NB_RING,)),
            pltpu.SemaphoreType.DMA((NB_RING,)),
        ],
    )
    return fn(emb, pair_tab, idx_nb, idx_pr, idx_h, dst_nb)


def _epilogue_body(feat_ref, h_ref, seq_ref, mask_ref, w_ref, out_ref):
    bb = seq_ref.shape[0]
    r = bb * L
    mask = mask_ref[...]
    seq = seq_ref[...]
    denom = jnp.sum(mask, axis=1, keepdims=True) + 1e-8
    s = jnp.sum(seq * mask[..., None], axis=1) / denom
    s_rep = jnp.broadcast_to(s[:, None, :], (bb, L, D)).reshape(r, D)

    feats = [feat_ref[n] for n in range(N)]
    w = w_ref[...]
    g = [jnp.sum(feats[n] * s_rep, axis=-1) * w[:, :, n].reshape(r)
         for n in range(N)]
    m = g[0]
    for n in range(1, N):
        m = jnp.maximum(m, g[n])
    u = [jnp.exp(g[n] - m) for n in range(N)]
    z = u[0]
    for n in range(1, N):
        z = z + u[n]
    inv = 1.0 / z
    agg = (u[0] * inv)[:, None] * feats[0]
    for n in range(1, N):
        agg = agg + (u[n] * inv)[:, None] * feats[n]
    out_ref[...] = jax.nn.relu(h_ref[...] + agg)


def _epilogue(featn, h2, seq, mask, wn, bb=8):
    grid = B // bb
    r = bb * L
    return pl.pallas_call(
        _epilogue_body,
        grid=(grid,),
        in_specs=[
            pl.BlockSpec((N, r, D), lambda p: (0, p, 0)),
            pl.BlockSpec((r, D), lambda p: (p, 0)),
            pl.BlockSpec((bb, L, D), lambda p: (p, 0, 0)),
            pl.BlockSpec((bb, L), lambda p: (p, 0)),
            pl.BlockSpec((bb, L, N), lambda p: (p, 0, 0)),
        ],
        out_specs=pl.BlockSpec((r, D), lambda p: (p, 0)),
        out_shape=jax.ShapeDtypeStruct((B * L, D), jnp.float32),
    )(featn, h2, seq, mask, wn)


def kernel(items, neighbors, weight_neighbors, seq_hidden_local, mask_item,
           pos_before_idx, pos_after_idx, embedding_table,
           pos_before_table, pos_after_table, pos_io_table):
    items = items.astype(jnp.int32)
    src = np.arange(NB_ROWS)
    dst_perm = ((src % N) * (B * L) + (src // (L * N)) * L
                + (src // N) % L).astype(np.int32)
    dst_nb = jnp.asarray(dst_perm.reshape(NW, NB_NCHUNK, NB_CHUNK))

    idx_nb = neighbors.astype(jnp.int32).reshape(NW, NB_NCHUNK, NB_CHUNK)
    pair_idx = (pos_before_idx.astype(jnp.int32) * P
                + pos_after_idx.astype(jnp.int32))
    idx_pr = pair_idx.reshape(NW, NB_NCHUNK, NB_CHUNK)
    idx_h = items.reshape(NW, H_NCHUNK, H_CHUNK)

    pair_tab = _build_pair_table(pos_before_table, pos_after_table,
                                 pos_io_table)
    feat, h = _sc_gather(embedding_table, pair_tab, idx_nb, idx_pr, idx_h,
                         dst_nb)

    out2 = _epilogue(feat.reshape(N, B * L, D), h,
                     seq_hidden_local, mask_item, weight_neighbors)
    return out2.reshape(B, L, D)


# speedup vs baseline: 1.0381x; 1.0381x over previous
import functools

import numpy as np
import jax
import jax.numpy as jnp
from jax import lax
from jax.experimental import pallas as pl
from jax.experimental.pallas import tpu as pltpu
import jax.experimental.pallas.tpu_sc as plsc

B, L, N, D, V, P = 1024, 50, 12, 64, 100000, 200

NW = 32
NB_ROWS = B * L * N
H_ROWS = B * L
NB_PER_W = NB_ROWS // NW
H_PER_W = H_ROWS // NW
NB_CHUNK = 120
NB_NCHUNK = NB_PER_W // NB_CHUNK
NB_RING = 8
NB_GROUPS = NB_NCHUNK // NB_RING
H_CHUNK = 80
H_NCHUNK = H_PER_W // H_CHUNK
H_RING = 4


def _pair_table_body(pb_ref, pa_ref, pio_ref, out_ref):
    comb = (pb_ref[...][:, None, :] + pa_ref[...][None, :, :]
            + pio_ref[1, :][None, None, :])
    out_ref[...] = comb.reshape(8 * P, D)


def _build_pair_table(pb, pa, pio):
    return pl.pallas_call(
        _pair_table_body,
        grid=(P // 8,),
        in_specs=[
            pl.BlockSpec((8, D), lambda i: (i, 0)),
            pl.BlockSpec((P, D), lambda i: (0, 0)),
            pl.BlockSpec((2, D), lambda i: (0, 0)),
        ],
        out_specs=pl.BlockSpec((8 * P, D), lambda i: (i, 0)),
        out_shape=jax.ShapeDtypeStruct((P * P, D), jnp.float32),
    )(pb, pa, pio)


def _sc_gather_body(emb_hbm, pair_hbm, idx_nb_hbm, idx_pr_hbm, idx_h_hbm,
                    feat_hbm, h_hbm,
                    idx_nb_v, idx_pr_v, idx_h_v, bufs, gsem, asem, wsem):
    wid = lax.axis_index("s") * 2 + lax.axis_index("c")

    pltpu.sync_copy(idx_nb_hbm.at[wid], idx_nb_v)
    pltpu.sync_copy(idx_pr_hbm.at[wid], idx_pr_v)
    pltpu.sync_copy(idx_h_hbm.at[wid], idx_h_v)

    nb_base = wid * NB_PER_W
    h_base = wid * H_PER_W

    def h_gather(c, b):
        return pltpu.async_copy(
            emb_hbm.at[idx_h_v.at[c]], bufs.at[b, pl.ds(0, H_CHUNK)],
            gsem.at[b])

    def h_write(c, b):
        return pltpu.async_copy(
            bufs.at[b, pl.ds(0, H_CHUNK)],
            h_hbm.at[pl.ds(h_base + c * H_CHUNK, H_CHUNK)],
            wsem.at[b])

    for c in range(H_RING):
        h_gather(c, c)
    for c in range(H_NCHUNK):
        b = c % H_RING
        pltpu.make_async_copy(emb_hbm.at[idx_h_v.at[c]],
                              bufs.at[b, pl.ds(0, H_CHUNK)], gsem.at[b]).wait()
        h_write(c, b)
        if c + H_RING < H_NCHUNK:
            pltpu.make_async_copy(bufs.at[b, pl.ds(0, H_CHUNK)],
                                  h_hbm.at[pl.ds(h_base + c * H_CHUNK, H_CHUNK)],
                                  wsem.at[b]).wait()
            h_gather(c + H_RING, b)
    for c in range(H_NCHUNK - H_RING, H_NCHUNK):
        b = c % H_RING
        pltpu.make_async_copy(bufs.at[b, pl.ds(0, H_CHUNK)],
                              h_hbm.at[pl.ds(h_base + c * H_CHUNK, H_CHUNK)],
                              wsem.at[b]).wait()

    def nb_gather(c, j):
        return pltpu.async_copy(emb_hbm.at[idx_nb_v.at[c]], bufs.at[j],
                                gsem.at[j])

    for j in range(NB_RING):
        nb_gather(j, j)

    def group(g, _):
        for j in range(NB_RING):
            c = g * NB_RING + j
            pltpu.make_async_copy(emb_hbm.at[idx_nb_v.at[c]], bufs.at[j],
                                  gsem.at[j]).wait()
            pltpu.async_copy(pair_hbm.at[idx_pr_v.at[c]], bufs.at[j],
                             asem.at[j], add=True)
        for j in range(NB_RING):
            c = g * NB_RING + j
            pltpu.make_async_copy(pair_hbm.at[idx_pr_v.at[c]], bufs.at[j],
                                  asem.at[j]).wait()
            pltpu.async_copy(bufs.at[j],
                             feat_hbm.at[pl.ds(nb_base + c * NB_CHUNK,
                                               NB_CHUNK)],
                             wsem.at[j])
        for j in range(NB_RING):
            c = g * NB_RING + j
            pltpu.make_async_copy(bufs.at[j],
                                  feat_hbm.at[pl.ds(nb_base + c * NB_CHUNK,
                                                    NB_CHUNK)],
                                  wsem.at[j]).wait()

            @pl.when(g < NB_GROUPS - 1)
            def _():
                nb_gather(g * NB_RING + NB_RING + j, j)

        return 0

    lax.fori_loop(0, NB_GROUPS, group, 0)


def _sc_gather(emb, pair_tab, idx_nb, idx_pr, idx_h):
    mesh = plsc.VectorSubcoreMesh(core_axis_name="c", subcore_axis_name="s")
    fn = pl.kernel(
        _sc_gather_body,
        out_type=(
            jax.ShapeDtypeStruct((NB_ROWS, D), jnp.float32),
            jax.ShapeDtypeStruct((H_ROWS, D), jnp.float32),
        ),
        mesh=mesh,
        compiler_params=pltpu.CompilerParams(use_tc_tiling_on_sc=False),
        scratch_types=[
            pltpu.VMEM((NB_NCHUNK, NB_CHUNK), jnp.int32),
            pltpu.VMEM((NB_NCHUNK, NB_CHUNK), jnp.int32),
            pltpu.VMEM((H_NCHUNK, H_CHUNK), jnp.int32),
            pltpu.VMEM((NB_RING, NB_CHUNK, D), jnp.float32),
            pltpu.SemaphoreType.DMA((NB_RING,)),
            pltpu.SemaphoreType.DMA((NB_RING,)),
            pltpu.SemaphoreType.DMA((NB_RING,)),
        ],
    )
    return fn(emb, pair_tab, idx_nb, idx_pr, idx_h)


def _epilogue_body(feat_ref, h_ref, seq_ref, mask_ref, w_ref, out_ref):
    bb = seq_ref.shape[0]
    r = bb * L
    mask = mask_ref[...]
    seq = seq_ref[...]
    denom = jnp.sum(mask, axis=1, keepdims=True) + 1e-8
    s = jnp.sum(seq * mask[..., None], axis=1) / denom
    s_rep = jnp.broadcast_to(s[:, None, :], (bb, L, D)).reshape(r, D)

    def tree(op, xs):
        while len(xs) > 1:
            xs = [op(xs[i], xs[i + 1]) if i + 1 < len(xs) else xs[i]
                  for i in range(0, len(xs), 2)]
        return xs[0]

    feats = [feat_ref[n] for n in range(N)]
    w = w_ref[...]
    g = [jnp.sum(feats[n] * s_rep, axis=-1) * w[:, :, n].reshape(r)
         for n in range(N)]
    m = tree(jnp.maximum, g)
    u = [jnp.exp(g[n] - m) for n in range(N)]
    z = tree(jnp.add, u)
    inv = 1.0 / z
    agg = tree(jnp.add, [(u[n] * inv)[:, None] * feats[n] for n in range(N)])
    out_ref[...] = jax.nn.relu(h_ref[...] + agg)


def _epilogue(featn, h2, seq, mask, wn, bb=8):
    grid = B // bb
    r = bb * L
    return pl.pallas_call(
        _epilogue_body,
        grid=(grid,),
        in_specs=[
            pl.BlockSpec((N, r, D), lambda p: (0, p, 0)),
            pl.BlockSpec((r, D), lambda p: (p, 0)),
            pl.BlockSpec((bb, L, D), lambda p: (p, 0, 0)),
            pl.BlockSpec((bb, L), lambda p: (p, 0)),
            pl.BlockSpec((bb, L, N), lambda p: (p, 0, 0)),
        ],
        out_specs=pl.BlockSpec((r, D), lambda p: (p, 0)),
        out_shape=jax.ShapeDtypeStruct((B * L, D), jnp.float32),
    )(featn, h2, seq, mask, wn)


def kernel(items, neighbors, weight_neighbors, seq_hidden_local, mask_item,
           pos_before_idx, pos_after_idx, embedding_table,
           pos_before_table, pos_after_table, pos_io_table):
    items = items.astype(jnp.int32)
    idx_nb = jnp.transpose(
        neighbors.astype(jnp.int32).reshape(B * L, N)
    ).reshape(NW, NB_NCHUNK, NB_CHUNK)
    pair_idx = (pos_before_idx.astype(jnp.int32) * P
                + pos_after_idx.astype(jnp.int32))
    idx_pr = jnp.transpose(pair_idx.reshape(B * L, N)).reshape(
        NW, NB_NCHUNK, NB_CHUNK)
    idx_h = items.reshape(NW, H_NCHUNK, H_CHUNK)

    pair_tab = _build_pair_table(pos_before_table, pos_after_table,
                                 pos_io_table)
    feat, h = _sc_gather(embedding_table, pair_tab, idx_nb, idx_pr, idx_h)

    out2 = _epilogue(feat.reshape(N, B * L, D), h,
                     seq_hidden_local, mask_item, weight_neighbors)
    return out2.reshape(B, L, D)


# speedup vs baseline: 1.0529x; 1.0143x over previous
import functools

import numpy as np
import jax
import jax.numpy as jnp
from jax import lax
from jax.experimental import pallas as pl
from jax.experimental.pallas import tpu as pltpu
import jax.experimental.pallas.tpu_sc as plsc

B, L, N, D, V, P = 1024, 50, 12, 64, 100000, 200

NW = 32
NB_CHUNK = 120
NB_RING = 8
H_CHUNK = 80
H_RING = 4


def _pair_table_body(pb_ref, pa_ref, pio_ref, out_ref):
    comb = (pb_ref[...][:, None, :] + pa_ref[...][None, :, :]
            + pio_ref[1, :][None, None, :])
    out_ref[...] = comb.reshape(8 * P, D)


def _build_pair_table(pb, pa, pio):
    return pl.pallas_call(
        _pair_table_body,
        grid=(P // 8,),
        in_specs=[
            pl.BlockSpec((8, D), lambda i: (i, 0)),
            pl.BlockSpec((P, D), lambda i: (0, 0)),
            pl.BlockSpec((2, D), lambda i: (0, 0)),
        ],
        out_specs=pl.BlockSpec((8 * P, D), lambda i: (i, 0)),
        out_shape=jax.ShapeDtypeStruct((P * P, D), jnp.float32),
    )(pb, pa, pio)


def _sc_gather_body(emb_hbm, pair_hbm, idx_nb_hbm, idx_pr_hbm, idx_h_hbm,
                    feat_hbm, h_hbm,
                    idx_nb_v, idx_pr_v, idx_h_v, bufs, gsem, asem, wsem):
    nb_nchunk = idx_nb_v.shape[0]
    nb_groups = nb_nchunk // NB_RING
    nb_per_w = nb_nchunk * NB_CHUNK
    h_nchunk = idx_h_v.shape[0]
    h_per_w = h_nchunk * H_CHUNK

    wid = lax.axis_index("s") * 2 + lax.axis_index("c")

    pltpu.sync_copy(idx_nb_hbm.at[wid], idx_nb_v)
    pltpu.sync_copy(idx_pr_hbm.at[wid], idx_pr_v)
    pltpu.sync_copy(idx_h_hbm.at[wid], idx_h_v)

    nb_base = wid * nb_per_w
    h_base = wid * h_per_w

    def h_gather(c, b):
        return pltpu.async_copy(
            emb_hbm.at[idx_h_v.at[c]], bufs.at[b, pl.ds(0, H_CHUNK)],
            gsem.at[b])

    def h_write(c, b):
        return pltpu.async_copy(
            bufs.at[b, pl.ds(0, H_CHUNK)],
            h_hbm.at[pl.ds(h_base + c * H_CHUNK, H_CHUNK)],
            wsem.at[b])

    for c in range(H_RING):
        h_gather(c, c)
    for c in range(h_nchunk):
        b = c % H_RING
        pltpu.make_async_copy(emb_hbm.at[idx_h_v.at[c]],
                              bufs.at[b, pl.ds(0, H_CHUNK)], gsem.at[b]).wait()
        h_write(c, b)
        if c + H_RING < h_nchunk:
            pltpu.make_async_copy(bufs.at[b, pl.ds(0, H_CHUNK)],
                                  h_hbm.at[pl.ds(h_base + c * H_CHUNK, H_CHUNK)],
                                  wsem.at[b]).wait()
            h_gather(c + H_RING, b)
    for c in range(h_nchunk - H_RING, h_nchunk):
        b = c % H_RING
        pltpu.make_async_copy(bufs.at[b, pl.ds(0, H_CHUNK)],
                              h_hbm.at[pl.ds(h_base + c * H_CHUNK, H_CHUNK)],
                              wsem.at[b]).wait()

    def nb_gather(c, j):
        return pltpu.async_copy(emb_hbm.at[idx_nb_v.at[c]], bufs.at[j],
                                gsem.at[j])

    for j in range(NB_RING):
        nb_gather(j, j)

    def group(g, _):
        for j in range(NB_RING):
            c = g * NB_RING + j
            pltpu.make_async_copy(emb_hbm.at[idx_nb_v.at[c]], bufs.at[j],
                                  gsem.at[j]).wait()
            pltpu.async_copy(pair_hbm.at[idx_pr_v.at[c]], bufs.at[j],
                             asem.at[j], add=True)
        for j in range(NB_RING):
            c = g * NB_RING + j
            pltpu.make_async_copy(pair_hbm.at[idx_pr_v.at[c]], bufs.at[j],
                                  asem.at[j]).wait()
            pltpu.async_copy(bufs.at[j],
                             feat_hbm.at[pl.ds(nb_base + c * NB_CHUNK,
                                               NB_CHUNK)],
                             wsem.at[j])
        for j in range(NB_RING):
            c = g * NB_RING + j
            pltpu.make_async_copy(bufs.at[j],
                                  feat_hbm.at[pl.ds(nb_base + c * NB_CHUNK,
                                                    NB_CHUNK)],
                                  wsem.at[j]).wait()

            @pl.when(g < nb_groups - 1)
            def _():
                nb_gather(g * NB_RING + NB_RING + j, j)

        return 0

    lax.fori_loop(0, nb_groups, group, 0)


def _sc_gather(emb, pair_tab, idx_nb, idx_pr, idx_h):
    nb_rows = NW * idx_nb.shape[1] * NB_CHUNK
    h_rows = NW * idx_h.shape[1] * H_CHUNK
    mesh = plsc.VectorSubcoreMesh(core_axis_name="c", subcore_axis_name="s")
    fn = pl.kernel(
        _sc_gather_body,
        out_type=(
            jax.ShapeDtypeStruct((nb_rows, D), jnp.float32),
            jax.ShapeDtypeStruct((h_rows, D), jnp.float32),
        ),
        mesh=mesh,
        compiler_params=pltpu.CompilerParams(use_tc_tiling_on_sc=False),
        scratch_types=[
            pltpu.VMEM((idx_nb.shape[1], NB_CHUNK), jnp.int32),
            pltpu.VMEM((idx_pr.shape[1], NB_CHUNK), jnp.int32),
            pltpu.VMEM((idx_h.shape[1], H_CHUNK), jnp.int32),
            pltpu.VMEM((NB_RING, NB_CHUNK, D), jnp.float32),
            pltpu.SemaphoreType.DMA((NB_RING,)),
            pltpu.SemaphoreType.DMA((NB_RING,)),
            pltpu.SemaphoreType.DMA((NB_RING,)),
        ],
    )
    return fn(emb, pair_tab, idx_nb, idx_pr, idx_h)


def _epilogue_body(feat_ref, h_ref, seq_ref, mask_ref, w_ref, out_ref):
    bb = seq_ref.shape[0]
    r = bb * L
    mask = mask_ref[...]
    seq = seq_ref[...]
    denom = jnp.sum(mask, axis=1, keepdims=True) + 1e-8
    s = jnp.sum(seq * mask[..., None], axis=1) / denom
    s_rep = jnp.broadcast_to(s[:, None, :], (bb, L, D)).reshape(r, D)

    def tree(op, xs):
        while len(xs) > 1:
            xs = [op(xs[i], xs[i + 1]) if i + 1 < len(xs) else xs[i]
                  for i in range(0, len(xs), 2)]
        return xs[0]

    feats = [feat_ref[n] for n in range(N)]
    w = w_ref[...]
    g = [jnp.sum(feats[n] * s_rep, axis=-1) * w[:, :, n].reshape(r)
         for n in range(N)]
    m = tree(jnp.maximum, g)
    u = [jnp.exp(g[n] - m) for n in range(N)]
    z = tree(jnp.add, u)
    inv = 1.0 / z
    agg = tree(jnp.add, [(u[n] * inv)[:, None] * feats[n] for n in range(N)])
    out_ref[...] = jax.nn.relu(h_ref[...] + agg)


def _epilogue(featn, h2, seq, mask, wn, bb=8):
    nb = seq.shape[0]
    grid = nb // bb
    r = bb * L
    return pl.pallas_call(
        _epilogue_body,
        grid=(grid,),
        in_specs=[
            pl.BlockSpec((N, r, D), lambda p: (0, p, 0)),
            pl.BlockSpec((r, D), lambda p: (p, 0)),
            pl.BlockSpec((bb, L, D), lambda p: (p, 0, 0)),
            pl.BlockSpec((bb, L), lambda p: (p, 0)),
            pl.BlockSpec((bb, L, N), lambda p: (p, 0, 0)),
        ],
        out_specs=pl.BlockSpec((r, D), lambda p: (p, 0)),
        out_shape=jax.ShapeDtypeStruct((nb * L, D), jnp.float32),
    )(featn, h2, seq, mask, wn)


HALVES = 2
BH = B // HALVES


def kernel(items, neighbors, weight_neighbors, seq_hidden_local, mask_item,
           pos_before_idx, pos_after_idx, embedding_table,
           pos_before_table, pos_after_table, pos_io_table):
    items = items.astype(jnp.int32)
    pair_idx = (pos_before_idx.astype(jnp.int32) * P
                + pos_after_idx.astype(jnp.int32))
    pair_tab = _build_pair_table(pos_before_table, pos_after_table,
                                 pos_io_table)

    nb_nchunk = (BH * L * N) // (NW * NB_CHUNK)
    h_nchunk = (BH * L) // (NW * H_CHUNK)

    outs = []
    for k in range(HALVES):
        sl = slice(k * BH, (k + 1) * BH)
        idx_nb = jnp.transpose(
            neighbors[sl].astype(jnp.int32).reshape(BH * L, N)
        ).reshape(NW, nb_nchunk, NB_CHUNK)
        idx_pr = jnp.transpose(pair_idx[sl].reshape(BH * L, N)).reshape(
            NW, nb_nchunk, NB_CHUNK)
        idx_h = items[sl].reshape(NW, h_nchunk, H_CHUNK)
        feat, h = _sc_gather(embedding_table, pair_tab, idx_nb, idx_pr, idx_h)
        out_k = _epilogue(feat.reshape(N, BH * L, D), h,
                          seq_hidden_local[sl], mask_item[sl],
                          weight_neighbors[sl])
        outs.append(out_k.reshape(BH, L, D))
    return jnp.concatenate(outs, axis=0)
